# baseline (speedup 1.0000x reference)
import functools

import jax
import jax.numpy as jnp
from jax.experimental import pallas as pl
from jax.experimental.pallas import tpu as pltpu


def _gmm_body(m_tiles_ref, experts_ref, starts_ref, ends_ref,
              x_ref, w_ref, o_ref, *, bt: int):
    w = pl.program_id(1)
    prev = m_tiles_ref[jnp.maximum(w - 1, 0)]
    first = jnp.logical_or(w == 0, m_tiles_ref[w] != prev)
    base = m_tiles_ref[w] * bt
    rows = base + jax.lax.broadcasted_iota(jnp.int32, (bt, 1), 0)
    mask = jnp.logical_and(rows >= starts_ref[w], rows < ends_ref[w])
    contrib = jax.lax.dot_general(
        x_ref[...], w_ref[0],
        dimension_numbers=(((1,), (1,)), ((), ())),
        preferred_element_type=jnp.float32)
    contrib = jnp.where(mask, contrib, 0.0)

    @pl.when(first)
    def _():
        o_ref[...] = contrib

    @pl.when(jnp.logical_not(first))
    def _():
        o_ref[...] += contrib


def kernel(input_tokens, weight, expert_assignments):
    t, d_in = input_tokens.shape
    e, d_out, _ = weight.shape

    bt = 128
    bn = 512
    m_tiles_total = t // bt
    n_tiles = d_out // bn
    num_work = m_tiles_total + e - 1

    a32 = expert_assignments.astype(jnp.int32)
    offsets = jnp.searchsorted(a32, jnp.arange(e + 1, dtype=jnp.int32),
                               side="left").astype(jnp.int32)
    sizes = offsets[1:] - offsets[:-1]
    first_tile = offsets[:-1] // bt
    last_tile = jnp.maximum(offsets[1:] - 1, 0) // bt
    tiles_per = jnp.where(sizes > 0, last_tile - first_tile + 1, 0)
    cum_incl = jnp.cumsum(tiles_per)
    cum_excl = cum_incl - tiles_per
    total = cum_incl[-1]

    wids = jnp.arange(num_work, dtype=jnp.int32)
    e_w = jnp.minimum(jnp.searchsorted(cum_incl, wids, side="right"),
                      e - 1).astype(jnp.int32)
    valid = wids < total
    m_w = jnp.where(valid, first_tile[e_w] + (wids - cum_excl[e_w]),
                    m_tiles_total - 1).astype(jnp.int32)
    starts = jnp.where(valid, jnp.maximum(offsets[e_w], m_w * bt),
                       0).astype(jnp.int32)
    ends = jnp.where(valid, jnp.minimum(offsets[e_w + 1], (m_w + 1) * bt),
                     0).astype(jnp.int32)

    grid_spec = pltpu.PrefetchScalarGridSpec(
        num_scalar_prefetch=4,
        grid=(n_tiles, num_work),
        in_specs=[
            pl.BlockSpec((bt, d_in),
                         lambda n, w, mt, ex, st, en: (mt[w], 0)),
            pl.BlockSpec((1, bn, d_in),
                         lambda n, w, mt, ex, st, en: (ex[w], n, 0)),
        ],
        out_specs=pl.BlockSpec((bt, bn),
                               lambda n, w, mt, ex, st, en: (mt[w], n)),
    )

    out = pl.pallas_call(
        functools.partial(_gmm_body, bt=bt),
        grid_spec=grid_spec,
        out_shape=jax.ShapeDtypeStruct((t, d_out), jnp.float32),
        compiler_params=pltpu.CompilerParams(
            dimension_semantics=("arbitrary", "arbitrary")),
    )(m_w, e_w, starts, ends, input_tokens, weight)
    return out


# speedup vs baseline: 2.2227x; 2.2227x over previous
import functools

import jax
import jax.numpy as jnp
from jax.experimental import pallas as pl
from jax.experimental.pallas import tpu as pltpu


def _gmm_body(m_tiles_ref, experts_ref, starts_ref, ends_ref,
              x_ref, w_ref, o_ref, *, bt: int):
    w = pl.program_id(1)
    prev = m_tiles_ref[jnp.maximum(w - 1, 0)]
    first = jnp.logical_or(w == 0, m_tiles_ref[w] != prev)
    base = m_tiles_ref[w] * bt
    rows = base + jax.lax.broadcasted_iota(jnp.int32, (bt, 1), 0)
    mask = jnp.logical_and(rows >= starts_ref[w], rows < ends_ref[w])
    contrib = jax.lax.dot_general(
        x_ref[...], w_ref[0],
        dimension_numbers=(((1,), (1,)), ((), ())),
        preferred_element_type=jnp.float32)
    contrib = jnp.where(mask, contrib, 0.0)

    @pl.when(first)
    def _():
        o_ref[...] = contrib

    @pl.when(jnp.logical_not(first))
    def _():
        o_ref[...] += contrib


def kernel(input_tokens, weight, expert_assignments):
    t, d_in = input_tokens.shape
    e, d_out, _ = weight.shape

    bt = 256
    bn = 1024
    m_tiles_total = t // bt
    n_tiles = d_out // bn
    num_work = m_tiles_total + e - 1

    a32 = expert_assignments.astype(jnp.int32)
    offsets = jnp.searchsorted(a32, jnp.arange(e + 1, dtype=jnp.int32),
                               side="left").astype(jnp.int32)
    sizes = offsets[1:] - offsets[:-1]
    first_tile = offsets[:-1] // bt
    last_tile = jnp.maximum(offsets[1:] - 1, 0) // bt
    tiles_per = jnp.where(sizes > 0, last_tile - first_tile + 1, 0)
    cum_incl = jnp.cumsum(tiles_per)
    cum_excl = cum_incl - tiles_per
    total = cum_incl[-1]

    wids = jnp.arange(num_work, dtype=jnp.int32)
    e_w = jnp.minimum(jnp.searchsorted(cum_incl, wids, side="right"),
                      e - 1).astype(jnp.int32)
    valid = wids < total
    m_w = jnp.where(valid, first_tile[e_w] + (wids - cum_excl[e_w]),
                    m_tiles_total - 1).astype(jnp.int32)
    starts = jnp.where(valid, jnp.maximum(offsets[e_w], m_w * bt),
                       0).astype(jnp.int32)
    ends = jnp.where(valid, jnp.minimum(offsets[e_w + 1], (m_w + 1) * bt),
                     0).astype(jnp.int32)

    grid_spec = pltpu.PrefetchScalarGridSpec(
        num_scalar_prefetch=4,
        grid=(n_tiles, num_work),
        in_specs=[
            pl.BlockSpec((bt, d_in),
                         lambda n, w, mt, ex, st, en: (mt[w], 0)),
            pl.BlockSpec((1, bn, d_in),
                         lambda n, w, mt, ex, st, en: (ex[w], n, 0)),
        ],
        out_specs=pl.BlockSpec((bt, bn),
                               lambda n, w, mt, ex, st, en: (mt[w], n)),
    )

    out = pl.pallas_call(
        functools.partial(_gmm_body, bt=bt),
        grid_spec=grid_spec,
        out_shape=jax.ShapeDtypeStruct((t, d_out), jnp.float32),
        compiler_params=pltpu.CompilerParams(
            dimension_semantics=("arbitrary", "arbitrary")),
    )(m_w, e_w, starts, ends, input_tokens, weight)
    return out


# speedup vs baseline: 2.6310x; 1.1837x over previous
import functools

import jax
import jax.numpy as jnp
from jax.experimental import pallas as pl
from jax.experimental.pallas import tpu as pltpu


def _gmm_body(m_tiles_ref, experts_ref, starts_ref, ends_ref,
              x_ref, w_ref, o_ref, *, bt: int):
    w = pl.program_id(1)
    prev = m_tiles_ref[jnp.maximum(w - 1, 0)]
    first = jnp.logical_or(w == 0, m_tiles_ref[w] != prev)
    base = m_tiles_ref[w] * bt
    rows = base + jax.lax.broadcasted_iota(jnp.int32, (bt, 1), 0)
    mask = jnp.logical_and(rows >= starts_ref[w], rows < ends_ref[w])
    contrib = jax.lax.dot_general(
        x_ref[...], w_ref[0],
        dimension_numbers=(((1,), (1,)), ((), ())),
        preferred_element_type=jnp.float32)
    contrib = jnp.where(mask, contrib, 0.0)

    @pl.when(first)
    def _():
        o_ref[...] = contrib

    @pl.when(jnp.logical_not(first))
    def _():
        o_ref[...] += contrib


def kernel(input_tokens, weight, expert_assignments):
    t, d_in = input_tokens.shape
    e, d_out, _ = weight.shape

    bt = 256
    bn = 2048
    m_tiles_total = t // bt
    n_tiles = d_out // bn
    num_work = m_tiles_total + e - 1

    a32 = expert_assignments.astype(jnp.int32)
    offsets = jnp.searchsorted(a32, jnp.arange(e + 1, dtype=jnp.int32),
                               side="left").astype(jnp.int32)
    sizes = offsets[1:] - offsets[:-1]
    first_tile = offsets[:-1] // bt
    last_tile = jnp.maximum(offsets[1:] - 1, 0) // bt
    tiles_per = jnp.where(sizes > 0, last_tile - first_tile + 1, 0)
    cum_incl = jnp.cumsum(tiles_per)
    cum_excl = cum_incl - tiles_per
    total = cum_incl[-1]

    wids = jnp.arange(num_work, dtype=jnp.int32)
    e_w = jnp.minimum(jnp.searchsorted(cum_incl, wids, side="right"),
                      e - 1).astype(jnp.int32)
    valid = wids < total
    m_w = jnp.where(valid, first_tile[e_w] + (wids - cum_excl[e_w]),
                    m_tiles_total - 1).astype(jnp.int32)
    starts = jnp.where(valid, jnp.maximum(offsets[e_w], m_w * bt),
                       0).astype(jnp.int32)
    ends = jnp.where(valid, jnp.minimum(offsets[e_w + 1], (m_w + 1) * bt),
                     0).astype(jnp.int32)

    grid_spec = pltpu.PrefetchScalarGridSpec(
        num_scalar_prefetch=4,
        grid=(n_tiles, num_work),
        in_specs=[
            pl.BlockSpec((bt, d_in),
                         lambda n, w, mt, ex, st, en: (mt[w], 0)),
            pl.BlockSpec((1, bn, d_in),
                         lambda n, w, mt, ex, st, en: (ex[w], n, 0)),
        ],
        out_specs=pl.BlockSpec((bt, bn),
                               lambda n, w, mt, ex, st, en: (mt[w], n)),
    )

    out = pl.pallas_call(
        functools.partial(_gmm_body, bt=bt),
        grid_spec=grid_spec,
        out_shape=jax.ShapeDtypeStruct((t, d_out), jnp.float32),
        compiler_params=pltpu.CompilerParams(
            dimension_semantics=("arbitrary", "arbitrary")),
    )(m_w, e_w, starts, ends, input_tokens, weight)
    return out
